# baseline (speedup 1.0000x reference)
import functools

import jax
import jax.numpy as jnp
from jax import lax
from jax.experimental import pallas as pl
from jax.experimental.pallas import tpu as pltpu
from jax.experimental.pallas import tpu_sc as plsc

_VOCAB = 8
_EMB_DIM = 30
_CONTEXT = 2


def kernel(x, emb, W, b):
    mesh = plsc.ScalarSubcoreMesh(axis_name="c", num_cores=1)

    @functools.partial(
        pl.kernel,
        mesh=mesh,
        out_type=jax.ShapeDtypeStruct((1, _VOCAB), jnp.float32),
        compiler_params=pltpu.CompilerParams(needs_layout_passes=False),
        scratch_types=[
            pltpu.SMEM((_CONTEXT,), jnp.int32),
            pltpu.SMEM((_VOCAB, _EMB_DIM), jnp.float32),
            pltpu.SMEM((_VOCAB, _EMB_DIM * _CONTEXT), jnp.float32),
            pltpu.SMEM((_VOCAB,), jnp.float32),
            pltpu.SMEM((_VOCAB,), jnp.float32),
            pltpu.SemaphoreType.DMA,
        ],
    )
    def sc_kernel(x_hbm, emb_hbm, w_hbm, b_hbm, out_hbm,
                  x_sm, emb_sm, w_sm, b_sm, out_sm, sem):
        @pl.when(lax.axis_index("c") == 0)
        def _():
            cx = pltpu.async_copy(x_hbm, x_sm, sem)
            ce = pltpu.async_copy(emb_hbm, emb_sm, sem)
            cw = pltpu.async_copy(w_hbm, w_sm, sem)
            cb = pltpu.async_copy(b_hbm, b_sm, sem)
            cx.wait()
            ce.wait()
            cw.wait()
            cb.wait()

            acc = tuple(b_sm[j] for j in range(_VOCAB))
            for c in range(_CONTEXT):
                xc = x_sm[c]

                def body(d, a, c=c, xc=xc):
                    h = emb_sm[xc, d]
                    return tuple(
                        a[j] + h * w_sm[j, c * _EMB_DIM + d]
                        for j in range(_VOCAB)
                    )

                acc = lax.fori_loop(0, _EMB_DIM, body, acc, unroll=5)
            for j in range(_VOCAB):
                out_sm[j] = acc[j]
            pltpu.sync_copy(out_sm, out_hbm.at[0])

    return sc_kernel(x, emb, W, b)


# speedup vs baseline: 1.0291x; 1.0291x over previous
import functools

import jax
import jax.numpy as jnp
from jax import lax
from jax.experimental import pallas as pl
from jax.experimental.pallas import tpu as pltpu
from jax.experimental.pallas import tpu_sc as plsc

_VOCAB = 8
_EMB_DIM = 30
_CONTEXT = 2


def kernel(x, emb, W, b):
    mesh = plsc.ScalarSubcoreMesh(axis_name="c", num_cores=1)

    @functools.partial(
        pl.kernel,
        mesh=mesh,
        out_type=jax.ShapeDtypeStruct((1, _VOCAB), jnp.float32),
        compiler_params=pltpu.CompilerParams(needs_layout_passes=False),
        scratch_types=[
            pltpu.SMEM((_CONTEXT,), jnp.int32),
            pltpu.SMEM((_VOCAB, _EMB_DIM), jnp.float32),
            pltpu.SMEM((_VOCAB, _EMB_DIM * _CONTEXT), jnp.float32),
            pltpu.SMEM((_VOCAB,), jnp.float32),
            pltpu.SMEM((_VOCAB,), jnp.float32),
            pltpu.SemaphoreType.DMA,
        ],
    )
    def sc_kernel(x_hbm, emb_hbm, w_hbm, b_hbm, out_hbm,
                  x_sm, emb_sm, w_sm, b_sm, out_sm, sem):
        def _():
            cx = pltpu.async_copy(x_hbm, x_sm, sem)
            ce = pltpu.async_copy(emb_hbm, emb_sm, sem)
            cw = pltpu.async_copy(w_hbm, w_sm, sem)
            cb = pltpu.async_copy(b_hbm, b_sm, sem)
            cx.wait()
            ce.wait()
            cw.wait()
            cb.wait()

            acc = tuple(b_sm[j] for j in range(_VOCAB))
            for c in range(_CONTEXT):
                xc = x_sm[c]

                def body(d, a, c=c, xc=xc):
                    h = emb_sm[xc, d]
                    return tuple(
                        a[j] + h * w_sm[j, c * _EMB_DIM + d]
                        for j in range(_VOCAB)
                    )

                acc = lax.fori_loop(0, _EMB_DIM, body, acc)
            for j in range(_VOCAB):
                out_sm[j] = acc[j]
            pltpu.sync_copy(out_sm, out_hbm.at[0])

        _()

    return sc_kernel(x, emb, W, b)


# speedup vs baseline: 1.0409x; 1.0115x over previous
import functools

import jax
import jax.numpy as jnp
from jax import lax
from jax.experimental import pallas as pl
from jax.experimental.pallas import tpu as pltpu
from jax.experimental.pallas import tpu_sc as plsc

_VOCAB = 8
_EMB_DIM = 30
_CONTEXT = 2


def kernel(x, emb, W, b):
    mesh = plsc.ScalarSubcoreMesh(axis_name="c", num_cores=1)

    @functools.partial(
        pl.kernel,
        mesh=mesh,
        out_type=jax.ShapeDtypeStruct((1, _VOCAB), jnp.float32),
        compiler_params=pltpu.CompilerParams(needs_layout_passes=False),
        scratch_types=[
            pltpu.SMEM((_CONTEXT,), jnp.int32),
            pltpu.SMEM((_VOCAB, _EMB_DIM), jnp.float32),
            pltpu.SMEM((_VOCAB, _EMB_DIM * _CONTEXT), jnp.float32),
            pltpu.SMEM((_VOCAB,), jnp.float32),
            pltpu.SMEM((_VOCAB,), jnp.float32),
            pltpu.SemaphoreType.DMA,
        ],
    )
    def sc_kernel(x_hbm, emb_hbm, w_hbm, b_hbm, out_hbm,
                  x_sm, emb_sm, w_sm, b_sm, out_sm, sem):
        def _():
            cx = pltpu.async_copy(x_hbm, x_sm, sem)
            ce = pltpu.async_copy(emb_hbm, emb_sm, sem)
            cw = pltpu.async_copy(w_hbm, w_sm, sem)
            cb = pltpu.async_copy(b_hbm, b_sm, sem)
            cx.wait()
            ce.wait()
            cw.wait()
            cb.wait()

            acc = tuple(b_sm[j] for j in range(_VOCAB))
            for c in range(_CONTEXT):
                xc = x_sm[c]

                def body(d, a, c=c, xc=xc):
                    h = emb_sm[xc, d]
                    return tuple(
                        a[j] + h * w_sm[j, c * _EMB_DIM + d]
                        for j in range(_VOCAB)
                    )

                acc = lax.fori_loop(0, _EMB_DIM, body, acc, unroll=2)
            for j in range(_VOCAB):
                out_sm[j] = acc[j]
            pltpu.sync_copy(out_sm, out_hbm.at[0])

        _()

    return sc_kernel(x, emb, W, b)
